# baseline (speedup 1.0000x reference)
import functools

import jax
import jax.numpy as jnp
import numpy as np
from jax.experimental import pallas as pl
from jax import lax
from jax.experimental.pallas import tpu as pltpu
from jax.experimental.pallas import tpu_sc as plsc

SR = 44100
NBINS = 128
HOP = 512
FMIN = 100.0
FMAX = 12800.0

BATCH = 4
NSAMP = 88200
T = 173
TPAD = 176
NB = 126
LANES = 128
NCHUNK = 32
ZROWS = 173
XROWS = 208
TSC = 192


def _make_consts():
    num_octaves = np.log2(FMAX / FMIN)
    bpo = int(NBINS / num_octaves)
    Q = 1.0 / (2.0 ** (1.0 / bpo) - 1.0)
    n_bins = int(np.ceil(bpo * np.log2(FMAX / FMIN)))
    freqs = FMIN * 2.0 ** (np.arange(n_bins, dtype=np.float64) / bpo)
    fft_len = int(2 ** np.ceil(np.log2(np.ceil(Q * SR / FMIN))))
    lengths = np.ceil(Q * SR / freqs)
    kr = np.zeros((n_bins, fft_len), dtype=np.float32)
    ki = np.zeros((n_bins, fft_len), dtype=np.float32)
    for k in range(n_bins):
        l = int(lengths[k])
        if l % 2 == 1:
            start = int(np.ceil(fft_len / 2.0 - l / 2.0)) - 1
        else:
            start = int(np.ceil(fft_len / 2.0 - l / 2.0))
        n = np.arange(l)
        win = 0.5 - 0.5 * np.cos(2.0 * np.pi * n / l)
        r = np.arange(-l // 2, -l // 2 + l)
        sig = (win / l) * np.exp(1j * 2.0 * np.pi * freqs[k] * r / SR)
        kr[k, start:start + l] = sig.real.astype(np.float32)
        ki[k, start:start + l] = sig.imag.astype(np.float32)
    kr3 = kr.reshape(n_bins, NCHUNK, HOP)
    ki3 = ki.reshape(n_bins, NCHUNK, HOP)
    nz = [c for c in range(NCHUNK)
          if (np.abs(kr3[:, c]).max() > 0) or (np.abs(ki3[:, c]).max() > 0)]
    kri = np.zeros((len(nz), HOP, 2 * LANES), dtype=np.float32)
    for i, c in enumerate(nz):
        kri[i, :, :n_bins] = kr3[:, c].T
        kri[i, :, LANES:LANES + n_bins] = ki3[:, c].T
    plm = np.zeros((HOP, HOP), dtype=np.float32)
    prm = np.zeros((HOP, HOP), dtype=np.float32)
    for l in range(HOP):
        plm[(HOP - l) % HOP, l] = 1.0
        prm[(270 - l) % HOP, l] = 1.0
    sql = np.zeros((1, LANES), dtype=np.float32)
    sql[0, :n_bins] = np.sqrt(lengths).astype(np.float32)
    cum = np.zeros((LANES, LANES), dtype=np.float32)
    for k in range(n_bins):
        cum[k, k:n_bins] = 1.0
    f = (freqs / SR).astype(np.float64)
    wd = f[1:] - f[:-1]
    w = np.zeros((1, LANES), dtype=np.float32)
    w[0, :n_bins - 1] = wd.astype(np.float32)
    wk = [float(v) for v in wd]
    wm = np.concatenate([[0.0], np.cumsum(wd)[:-1]])
    p = np.zeros((1, LANES), dtype=np.float32)
    p[0, :n_bins - 1] = (wd * (2.0 * wm + wd)).astype(np.float32)
    return (jnp.asarray(kri), jnp.asarray(plm), jnp.asarray(prm),
            jnp.asarray(sql), jnp.asarray(cum), jnp.asarray(w),
            jnp.asarray(p), wk, fft_len, nz)


(_KRI, _PL, _PR, _SQL, _CUM, _W, _P, _WK, _FFTLEN, _NZ) = _make_consts()
_PAD = _FFTLEN // 2


def _body(xzx_ref, xzy_ref, kri_ref, pl_ref, pr_ref, sql_ref, cum_ref,
          out_ref, xs1_ref, xs2_ref):
    dn = (((1,), (0,)), ((), ()))
    li = jax.lax.broadcasted_iota(jnp.int32, (1, HOP), 1)

    def fill(xz, xs_ref):
        pla = jax.lax.dot_general(xz[0:24, :], pl_ref[...], dn,
                                  preferred_element_type=jnp.float32)
        pra = jax.lax.dot_general(xz[152:173, :], pr_ref[...], dn,
                                  preferred_element_type=jnp.float32)
        xs_ref[16:16 + ZROWS, :] = xz
        for j in range(5, 16):
            xs_ref[j:j + 1, :] = jnp.where(li == 0, pla[16 - j:17 - j, :],
                                           pla[15 - j:16 - j, :])
        for j in range(188, 202):
            row = jnp.where(li <= 270, pra[208 - j:209 - j, :],
                            pra[207 - j:208 - j, :])
            if j == 188:
                row = jnp.where(li < 136, xz[ZROWS - 1:ZROWS, :], row)
            xs_ref[j:j + 1, :] = row

    def finish(acc):
        acc_r = acc[:, :LANES]
        acc_i = acc[:, LANES:]
        mag = jnp.sqrt(acc_r * acc_r + acc_i * acc_i) * sql_ref[...]
        fx = jnp.log(mag + 1.0)
        F = jax.lax.dot_general(fx, cum_ref[...], dn,
                                preferred_element_type=jnp.float32)
        return F / F[:, NB - 1:NB]

    fill(xzx_ref[0], xs1_ref)
    fill(xzy_ref[0], xs2_ref)
    acc_a = jnp.zeros((TPAD, 2 * LANES), jnp.float32)
    acc_b = jnp.zeros((TPAD, 2 * LANES), jnp.float32)
    for i, c in enumerate(_NZ):
        acc_a += jax.lax.dot_general(xs1_ref[c:c + TPAD, :], kri_ref[i],
                                     dn, preferred_element_type=jnp.float32)
        acc_b += jax.lax.dot_general(xs2_ref[c:c + TPAD, :], kri_ref[i],
                                     dn, preferred_element_type=jnp.float32)
    a = finish(acc_a)
    b = finish(acc_b)
    ti = jax.lax.broadcasted_iota(jnp.int32, (TPAD, 1), 0)
    zpad = jnp.zeros((TSC - TPAD, LANES), jnp.float32)
    out_ref[0, 0, 0:TPAD, :] = jnp.where(ti < T, a, 0.0)
    out_ref[0, 0, TPAD:TSC, :] = zpad
    out_ref[1, 0, 0:TPAD, :] = jnp.where(ti < T, b, 0.0)
    out_ref[1, 0, TPAD:TSC, :] = zpad


@jax.jit
def kernel(y, x):
    xy = jnp.pad(jnp.concatenate([x, y], axis=0),
                 ((0, 0), (0, ZROWS * HOP - NSAMP)))
    xy = xy.reshape(2 * BATCH, ZROWS, HOP)

    out = pl.pallas_call(
        _body,
        grid=(BATCH,),
        in_specs=[
            pl.BlockSpec((1, ZROWS, HOP), lambda b: (b, 0, 0)),
            pl.BlockSpec((1, ZROWS, HOP), lambda b: (b + BATCH, 0, 0)),
            pl.BlockSpec(_KRI.shape, lambda b: (0, 0, 0)),
            pl.BlockSpec((HOP, HOP), lambda b: (0, 0)),
            pl.BlockSpec((HOP, HOP), lambda b: (0, 0)),
            pl.BlockSpec((1, LANES), lambda b: (0, 0)),
            pl.BlockSpec((LANES, LANES), lambda b: (0, 0)),
        ],
        out_specs=pl.BlockSpec((2, 1, TSC, LANES), lambda b: (0, b, 0, 0)),
        out_shape=jax.ShapeDtypeStruct((2, BATCH, TSC, LANES), jnp.float32),
        scratch_shapes=[pltpu.VMEM((XROWS, HOP), jnp.float32),
                        pltpu.VMEM((XROWS, HOP), jnp.float32)],
    )(xy, xy, _KRI, _PL, _PR, _SQL, _CUM)

    cdf = out.reshape(2, BATCH * TSC, LANES)
    parts = _sc_ot(cdf[0], cdf[1], _WV, _PV)

    fin = pl.pallas_call(
        _fin_body,
        grid=(1,),
        in_specs=[pl.BlockSpec((_NW, 8, 16), lambda i: (0, 0, 0))],
        out_specs=pl.BlockSpec((BATCH, 1), lambda i: (0, 0),
                               memory_space=pltpu.SMEM),
        out_shape=jax.ShapeDtypeStruct((BATCH, 1), jnp.float32),
    )(parts)

    return fin.reshape(BATCH)


_NW = 32
_CPW = BATCH * TSC // _NW
_WV = _W.reshape(LANES)
_PV = _P.reshape(LANES)


@functools.partial(
    pl.kernel,
    mesh=plsc.VectorSubcoreMesh(core_axis_name="c", subcore_axis_name="s"),
    out_type=jax.ShapeDtypeStruct((_NW, 8, 16), jnp.float32),
    scratch_types=[
        pltpu.VMEM((_CPW, LANES), jnp.float32),
        pltpu.VMEM((_CPW, LANES), jnp.float32),
        pltpu.VMEM((LANES,), jnp.float32),
        pltpu.VMEM((LANES,), jnp.float32),
        pltpu.VMEM((8, 16), jnp.float32),
    ],
)
def _sc_ot(a_hbm, b_hbm, w_hbm, p_hbm, out_hbm, av, bv, wv, pv, res):
    wid = lax.axis_index("s") * 2 + lax.axis_index("c")
    base = wid * _CPW
    pltpu.sync_copy(a_hbm.at[pl.ds(base, _CPW)], av)
    pltpu.sync_copy(b_hbm.at[pl.ds(base, _CPW)], bv)
    pltpu.sync_copy(w_hbm, wv)
    pltpu.sync_copy(p_hbm, pv)

    def col(r, total):
        accs = [jnp.zeros((16,), jnp.float32) for _ in range(8)]
        b16 = [bv[r, pl.ds(j * 16, 16)] for j in range(8)]
        for kc in range(8):
            a16 = av[r, pl.ds(kc * 16, 16)]
            for kk in range(16):
                k = kc * 16 + kk
                if k >= NB - 1:
                    break
                ak = a16[kk]
                for j in range(8):
                    accs[j] = accs[j] + _WK[k] * jnp.maximum(ak, b16[j])
        csum = jnp.zeros((16,), jnp.float32)
        for j in range(8):
            sl = pl.ds(j * 16, 16)
            csum = csum + 2.0 * accs[j] * wv[sl] \
                - pv[sl] * (av[r, sl] + b16[j])
        return total + csum

    total = lax.fori_loop(0, _CPW, col, jnp.zeros((16,), jnp.float32))
    res[0, :] = total
    pltpu.sync_copy(res, out_hbm.at[wid])


def _fin_body(parts_ref, out_ref):
    m = parts_ref[...]
    for b in range(BATCH):
        out_ref[b, 0] = jnp.sum(m[8 * b:8 * b + 8, 0, :]) * (100.0 / T)


# speedup vs baseline: 1.6730x; 1.6730x over previous
import jax
import jax.numpy as jnp
import numpy as np
from jax.experimental import pallas as pl
from jax.experimental.pallas import tpu as pltpu

SR = 44100
NBINS = 128
HOP = 512
FMIN = 100.0
FMAX = 12800.0

BATCH = 4
NSAMP = 88200
T = 173
TPAD = 176
NB = 126
LANES = 128
NCHUNK = 32
ZROWS = 173
XROWS = 208


def _make_consts():
    num_octaves = np.log2(FMAX / FMIN)
    bpo = int(NBINS / num_octaves)
    Q = 1.0 / (2.0 ** (1.0 / bpo) - 1.0)
    n_bins = int(np.ceil(bpo * np.log2(FMAX / FMIN)))
    freqs = FMIN * 2.0 ** (np.arange(n_bins, dtype=np.float64) / bpo)
    fft_len = int(2 ** np.ceil(np.log2(np.ceil(Q * SR / FMIN))))
    lengths = np.ceil(Q * SR / freqs)
    kr = np.zeros((n_bins, fft_len), dtype=np.float32)
    ki = np.zeros((n_bins, fft_len), dtype=np.float32)
    for k in range(n_bins):
        l = int(lengths[k])
        if l % 2 == 1:
            start = int(np.ceil(fft_len / 2.0 - l / 2.0)) - 1
        else:
            start = int(np.ceil(fft_len / 2.0 - l / 2.0))
        n = np.arange(l)
        win = 0.5 - 0.5 * np.cos(2.0 * np.pi * n / l)
        r = np.arange(-l // 2, -l // 2 + l)
        sig = (win / l) * np.exp(1j * 2.0 * np.pi * freqs[k] * r / SR)
        kr[k, start:start + l] = sig.real.astype(np.float32)
        ki[k, start:start + l] = sig.imag.astype(np.float32)
    kr3 = kr.reshape(n_bins, NCHUNK, HOP)
    ki3 = ki.reshape(n_bins, NCHUNK, HOP)
    nz = [c for c in range(NCHUNK)
          if (np.abs(kr3[:, c]).max() > 0) or (np.abs(ki3[:, c]).max() > 0)]
    kri = np.zeros((len(nz), HOP, 2 * LANES), dtype=np.float32)
    for i, c in enumerate(nz):
        kri[i, :, :n_bins] = kr3[:, c].T
        kri[i, :, LANES:LANES + n_bins] = ki3[:, c].T
    plm = np.zeros((HOP, HOP), dtype=np.float32)
    prm = np.zeros((HOP, HOP), dtype=np.float32)
    for l in range(HOP):
        plm[(HOP - l) % HOP, l] = 1.0
        prm[(270 - l) % HOP, l] = 1.0
    sql = np.zeros((1, LANES), dtype=np.float32)
    sql[0, :n_bins] = np.sqrt(lengths).astype(np.float32)
    cum = np.zeros((LANES, LANES), dtype=np.float32)
    for k in range(n_bins):
        cum[k, k:n_bins] = 1.0
    f = (freqs / SR).astype(np.float64)
    wd = f[1:] - f[:-1]
    w = np.zeros((1, LANES), dtype=np.float32)
    w[0, :n_bins - 1] = wd.astype(np.float32)
    wk = [float(v) for v in wd]
    wm = np.concatenate([[0.0], np.cumsum(wd)[:-1]])
    p = np.zeros((1, LANES), dtype=np.float32)
    p[0, :n_bins - 1] = (wd * (2.0 * wm + wd)).astype(np.float32)
    return (jnp.asarray(kri), jnp.asarray(plm), jnp.asarray(prm),
            jnp.asarray(sql), jnp.asarray(cum), jnp.asarray(w),
            jnp.asarray(p), wk, fft_len, nz)


(_KRI, _PL, _PR, _SQL, _CUM, _W, _P, _WK, _FFTLEN, _NZ) = _make_consts()
_PAD = _FFTLEN // 2


def _body(xzx_ref, xzy_ref, kri_ref, pl_ref, pr_ref, sql_ref, cum_ref,
          w_ref, p_ref, out_ref, xs1_ref, xs2_ref):
    dn = (((1,), (0,)), ((), ()))
    li = jax.lax.broadcasted_iota(jnp.int32, (1, HOP), 1)

    def fill(xz, xs_ref):
        pla = jax.lax.dot_general(xz[0:24, :], pl_ref[...], dn,
                                  preferred_element_type=jnp.float32)
        pra = jax.lax.dot_general(xz[152:173, :], pr_ref[...], dn,
                                  preferred_element_type=jnp.float32)
        xs_ref[16:16 + ZROWS, :] = xz
        for j in range(5, 16):
            xs_ref[j:j + 1, :] = jnp.where(li == 0, pla[16 - j:17 - j, :],
                                           pla[15 - j:16 - j, :])
        for j in range(188, 202):
            row = jnp.where(li <= 270, pra[208 - j:209 - j, :],
                            pra[207 - j:208 - j, :])
            if j == 188:
                row = jnp.where(li < 136, xz[ZROWS - 1:ZROWS, :], row)
            xs_ref[j:j + 1, :] = row

    def finish(acc):
        acc_r = acc[:, :LANES]
        acc_i = acc[:, LANES:]
        mag = jnp.sqrt(acc_r * acc_r + acc_i * acc_i) * sql_ref[...]
        fx = jnp.log(mag + 1.0)
        F = jax.lax.dot_general(fx, cum_ref[...], dn,
                                preferred_element_type=jnp.float32)
        return F / F[:, NB - 1:NB]

    fill(xzx_ref[0], xs1_ref)
    fill(xzy_ref[0], xs2_ref)
    acc_a = jnp.zeros((TPAD, 2 * LANES), jnp.float32)
    acc_b = jnp.zeros((TPAD, 2 * LANES), jnp.float32)
    for i, c in enumerate(_NZ):
        acc_a += jax.lax.dot_general(xs1_ref[c:c + TPAD, :], kri_ref[i],
                                     dn, preferred_element_type=jnp.float32)
        acc_b += jax.lax.dot_general(xs2_ref[c:c + TPAD, :], kri_ref[i],
                                     dn, preferred_element_type=jnp.float32)
    a = finish(acc_a)
    b = finish(acc_b)
    acc = jnp.zeros((TPAD, LANES), jnp.float32)
    for k in range(NB - 1):
        acc += _WK[k] * jnp.maximum(a[:, k:k + 1], b)
    r = 2.0 * acc * w_ref[...] - p_ref[...] * (a + b)
    ti = jax.lax.broadcasted_iota(jnp.int32, (TPAD, 1), 0)
    rm = jnp.where(ti < T, r, 0.0)
    out_ref[pl.program_id(0), 0] = jnp.sum(rm) * (100.0 / T)


@jax.jit
def kernel(y, x):
    xy = jnp.pad(jnp.concatenate([x, y], axis=0),
                 ((0, 0), (0, ZROWS * HOP - NSAMP)))
    xy = xy.reshape(2 * BATCH, ZROWS, HOP)

    out = pl.pallas_call(
        _body,
        grid=(BATCH,),
        in_specs=[
            pl.BlockSpec((1, ZROWS, HOP), lambda b: (b, 0, 0)),
            pl.BlockSpec((1, ZROWS, HOP), lambda b: (b + BATCH, 0, 0)),
            pl.BlockSpec(_KRI.shape, lambda b: (0, 0, 0)),
            pl.BlockSpec((HOP, HOP), lambda b: (0, 0)),
            pl.BlockSpec((HOP, HOP), lambda b: (0, 0)),
            pl.BlockSpec((1, LANES), lambda b: (0, 0)),
            pl.BlockSpec((LANES, LANES), lambda b: (0, 0)),
            pl.BlockSpec((1, LANES), lambda b: (0, 0)),
            pl.BlockSpec((1, LANES), lambda b: (0, 0)),
        ],
        out_specs=pl.BlockSpec((BATCH, 1), lambda b: (0, 0),
                               memory_space=pltpu.SMEM),
        out_shape=jax.ShapeDtypeStruct((BATCH, 1), jnp.float32),
        scratch_shapes=[pltpu.VMEM((XROWS, HOP), jnp.float32),
                        pltpu.VMEM((XROWS, HOP), jnp.float32)],
    )(xy, xy, _KRI, _PL, _PR, _SQL, _CUM, _W, _P)

    return out.reshape(BATCH)
